# baseline (speedup 1.0000x reference)
import math

import jax
import jax.numpy as jnp
from jax.experimental import pallas as pl

D = 128
KNOWN = 2048
BATCH = 1024
D_TREND = 16
TOP_K = 64
_ISQ = 1.0 / math.sqrt(D)
_I32MIN = -2147483648
_I32MAX = 2147483647


def _prelu(x, a_vec):
    return jnp.maximum(x, 0.0) + a_vec * jnp.minimum(x, 0.0)


def _prep_body(x_k, pe_k, x_u, pe_u,
               arw1, arw2, arw3, ab1, ab2, ab3, a_ar,
               pgw1, pgw12, pgw2, pgw3, pb1, pb12, pb2, pb3, gamma, beta,
               wq, wk,
               q_k_out, k_k_out, q_u_out):
    def dot(a, b):
        return jnp.dot(a, b, preferred_element_type=jnp.float32)

    def group(x_ref, pe_ref):
        x = x_ref[...]
        pe = pe_ref[...]
        h = _prelu(dot(x, arw1[...]) + ab1[...], a_ar[...])
        h = _prelu(dot(h, arw2[...]) + ab2[...], a_ar[...])
        h = dot(h, arw3[...]) + ab3[...]
        t1 = dot(h, pgw1[...]) + pb1[...] + dot(pe, pgw12[...]) + pb12[...]
        z = (dot(t1, pgw2[...]) + pb2[...]) * jax.nn.sigmoid(
            dot(t1, pgw3[...]) + pb3[...]) + h
        m = jnp.mean(z, axis=0, keepdims=True)
        v = jnp.mean((z - m) ** 2, axis=0, keepdims=True)
        ae = gamma[...] * (z - m) / jnp.sqrt(v + 1e-5) + beta[...]
        inp = 0.5 * ae + 0.5 * pe
        return inp

    in_k = group(x_k, pe_k)
    q_k_out[...] = dot(in_k, wq[...]) + in_k
    k_k_out[...] = dot(in_k, wk[...]) + in_k
    in_u = group(x_u, pe_u)
    q_u_out[...] = dot(in_u, wq[...]) + in_u


def _prep(x_k, pe_k, x_u, pe_u, weights):
    outs = [
        jax.ShapeDtypeStruct((KNOWN, D), jnp.float32),
        jax.ShapeDtypeStruct((KNOWN, D), jnp.float32),
        jax.ShapeDtypeStruct((BATCH, D), jnp.float32),
    ]
    return pl.pallas_call(
        _prep_body,
        out_shape=outs,
    )(x_k, pe_k, x_u, pe_u, *weights)


def _att_body(q_ref, peq_ref, key_ref, pekv_ref, out_ref, *, bq):
    nt = (((1,), (1,)), ((), ()))
    att = jax.lax.dot_general(q_ref[...], key_ref[...], nt,
                              preferred_element_type=jnp.float32) * _ISQ
    sims = jax.lax.dot_general(peq_ref[...], pekv_ref[...], nt,
                               preferred_element_type=jnp.float32)
    u = jax.lax.bitcast_convert_type(sims, jnp.int32)
    g = jnp.where(u >= 0, u, u ^ jnp.int32(0x7FFFFFFF))

    lo0 = jnp.min(g, axis=1, keepdims=True)
    hi0 = jnp.max(g, axis=1, keepdims=True) + 1
    cl0 = jnp.full((bq, 1), KNOWN, jnp.int32)
    ones_w = jnp.ones((KNOWN, D), jnp.bfloat16)

    def cond(c):
        lo, hi, cl = c
        return jnp.any((hi > lo + 1) & (cl != TOP_K))

    def body(c):
        lo, hi, cl = c
        mid = (lo & hi) + ((lo ^ hi) >> 1)
        onesb = jnp.where(g >= mid, 1.0, 0.0).astype(jnp.bfloat16)
        cnt = jnp.dot(onesb, ones_w,
                      preferred_element_type=jnp.float32)[:, 0:1]
        cnt = cnt.astype(jnp.int32)
        pred = cnt >= TOP_K
        return (jnp.where(pred, mid, lo),
                jnp.where(pred, hi, mid),
                jnp.where(pred, cnt, cl))

    lo, _, _ = jax.lax.while_loop(cond, body, (lo0, hi0, cl0))
    out_ref[...] = jnp.where(g >= lo, att, 0.0)


def _attention(q, pe_q, key, pe_kv, bq):
    nq = q.shape[0]
    grid = (nq // bq,)
    import functools
    body = functools.partial(_att_body, bq=bq)
    return pl.pallas_call(
        body,
        grid=grid,
        in_specs=[
            pl.BlockSpec((bq, D), lambda i: (i, 0)),
            pl.BlockSpec((bq, D), lambda i: (i, 0)),
            pl.BlockSpec((KNOWN, D), lambda i: (0, 0)),
            pl.BlockSpec((KNOWN, D), lambda i: (0, 0)),
        ],
        out_specs=pl.BlockSpec((bq, KNOWN), lambda i: (i, 0)),
        out_shape=jax.ShapeDtypeStruct((nq, KNOWN), jnp.float32),
    )(q, pe_q, key, pe_kv)


def _metapn_body(pe_ref, cd_ref,
                 w1t, w1b, b1wt, b1b,
                 w2t, w2bm, b2wt, b2b,
                 w3pt, w3bp, b3wt, b3b, a_vec,
                 out_ref, *, bm):
    def dot(a, b):
        return jnp.dot(a, b, preferred_element_type=jnp.float32)

    pe = pe_ref[...]
    cd = cd_ref[...]
    av = a_vec[...]
    pe_b = pe.astype(jnp.bfloat16)

    g1 = dot(pe, w1t[...]) + w1b[...]
    b1r = dot(pe, b1wt[...]) + b1b[...]
    x1 = _prelu(cd[:, 0:1] * g1[:, :D] + cd[:, 1:2] * g1[:, D:] + b1r, av)

    acc = dot(pe, b2wt[...]) + b2b[...] + dot(x1, w2bm[...])
    for kc in range(8):
        g2c = dot(pe_b, w2t[:, kc * 2048:(kc + 1) * 2048])
        for j in range(16):
            k = kc * 16 + j
            acc = acc + x1[:, k:k + 1] * g2c[:, j * D:(j + 1) * D]
    x2 = _prelu(acc, av)

    g3 = dot(pe_b, w3pt[...]) + w3bp[...]
    b3r = dot(pe, b3wt[...]) + b3b[...]
    cols = [jnp.sum(x2 * g3[:, t * D:(t + 1) * D], axis=1, keepdims=True)
            for t in range(D_TREND)]
    cols.append(jnp.zeros((bm, D - D_TREND), jnp.float32))
    out_ref[...] = jnp.concatenate(cols, axis=1) + b3r


def _metapn(pe_all, cd_pad, weights, bm):
    n = pe_all.shape[0]
    import functools
    body = functools.partial(_metapn_body, bm=bm)
    wspecs = [pl.BlockSpec(w.shape, lambda i: tuple(0 for _ in w.shape))
              for w in weights]
    return pl.pallas_call(
        body,
        grid=(n // bm,),
        in_specs=[
            pl.BlockSpec((bm, D), lambda i: (i, 0)),
            pl.BlockSpec((bm, D), lambda i: (i, 0)),
        ] + wspecs,
        out_specs=pl.BlockSpec((bm, D), lambda i: (i, 0)),
        out_shape=jax.ShapeDtypeStruct((n, D), jnp.float32),
    )(pe_all, cd_pad, *weights)


def kernel(x_know, x_unknow, pe_know, pe_unknow, coods_know, coods_unknow,
           params):
    p = params
    f32 = jnp.float32
    row = lambda v: v.reshape(1, -1).astype(f32)

    a_ar = jnp.full((1, D), p['ar_a'], f32)
    prep_w = (
        p['ar_w1'].T, p['ar_w2'].T, p['ar_w3'].T,
        row(p['ar_b1']), row(p['ar_b2']), row(p['ar_b3']), a_ar,
        p['pg_w1'].T, p['pg_w12'].T, p['pg_w2'].T, p['pg_w3'].T,
        row(p['pg_b1']), row(p['pg_b12']), row(p['pg_b2']), row(p['pg_b3']),
        row(p['pg_gamma']), row(p['pg_beta']),
        p['ss_wq'].T, p['ss_wk'].T,
    )
    q_k, k_k, q_u = _prep(x_know, pe_know, x_unknow, pe_unknow, prep_w)

    cov_know = _attention(q_k, pe_know, k_k, pe_know, bq=256)
    cov_unknow = _attention(q_u, pe_unknow, k_k, pe_know, bq=256)

    w3p = p['mp_w3w'].reshape(D, D_TREND, D).transpose(1, 0, 2).reshape(
        D_TREND * D, D)
    w3bp = p['mp_w3b'].reshape(D, D_TREND).T.reshape(1, D_TREND * D)
    b3wt = jnp.zeros((D, D), f32).at[:, :D_TREND].set(p['mp_b3w'].T)
    b3b = jnp.zeros((1, D), f32).at[:, :D_TREND].set(p['mp_b3b'].reshape(1, -1))
    a_mp = jnp.full((1, D), p['mp_a'], f32)
    mp_w = (
        p['mp_w1w'].T, row(p['mp_w1b']),
        p['mp_b1w'].T, row(p['mp_b1b']),
        p['mp_w2w'].T.astype(jnp.bfloat16), p['mp_w2b'].reshape(D, D),
        p['mp_b2w'].T, row(p['mp_b2b']),
        w3p.T.astype(jnp.bfloat16), w3bp,
        b3wt, b3b, a_mp,
    )
    pe_all = jnp.concatenate([pe_know, pe_unknow], axis=0)
    cd_all = jnp.concatenate([coods_know, coods_unknow], axis=0)
    cd_pad = jnp.pad(cd_all, ((0, 0), (0, D - 2)))
    trend = _metapn(pe_all, cd_pad, mp_w, bm=256)

    return (cov_know, cov_unknow,
            trend[:KNOWN, :D_TREND], trend[KNOWN:, :D_TREND])


# speedup vs baseline: 1.2615x; 1.2615x over previous
import math

import jax
import jax.numpy as jnp
from jax.experimental import pallas as pl

D = 128
KNOWN = 2048
BATCH = 1024
D_TREND = 16
TOP_K = 64
_ISQ = 1.0 / math.sqrt(D)
_I32MIN = -2147483648
_I32MAX = 2147483647


def _prelu(x, a_vec):
    return jnp.maximum(x, 0.0) + a_vec * jnp.minimum(x, 0.0)


def _prep_body(x_k, pe_k, x_u, pe_u,
               arw1, arw2, arw3, ab1, ab2, ab3, a_ar,
               pgw1, pgw12, pgw2, pgw3, pb1, pb12, pb2, pb3, gamma, beta,
               wq, wk,
               q_k_out, k_k_out, q_u_out):
    def dot(a, b):
        return jnp.dot(a, b, preferred_element_type=jnp.float32)

    def group(x_ref, pe_ref):
        x = x_ref[...]
        pe = pe_ref[...]
        h = _prelu(dot(x, arw1[...]) + ab1[...], a_ar[...])
        h = _prelu(dot(h, arw2[...]) + ab2[...], a_ar[...])
        h = dot(h, arw3[...]) + ab3[...]
        t1 = dot(h, pgw1[...]) + pb1[...] + dot(pe, pgw12[...]) + pb12[...]
        z = (dot(t1, pgw2[...]) + pb2[...]) * jax.nn.sigmoid(
            dot(t1, pgw3[...]) + pb3[...]) + h
        m = jnp.mean(z, axis=0, keepdims=True)
        v = jnp.mean((z - m) ** 2, axis=0, keepdims=True)
        ae = gamma[...] * (z - m) / jnp.sqrt(v + 1e-5) + beta[...]
        inp = 0.5 * ae + 0.5 * pe
        return inp

    in_k = group(x_k, pe_k)
    q_k_out[...] = dot(in_k, wq[...]) + in_k
    k_k_out[...] = dot(in_k, wk[...]) + in_k
    in_u = group(x_u, pe_u)
    q_u_out[...] = dot(in_u, wq[...]) + in_u


def _prep(x_k, pe_k, x_u, pe_u, weights):
    outs = [
        jax.ShapeDtypeStruct((KNOWN, D), jnp.float32),
        jax.ShapeDtypeStruct((KNOWN, D), jnp.float32),
        jax.ShapeDtypeStruct((BATCH, D), jnp.float32),
    ]
    return pl.pallas_call(
        _prep_body,
        out_shape=outs,
    )(x_k, pe_k, x_u, pe_u, *weights)


def _att_body(q_ref, peq_ref, key_ref, pekv_ref, out_ref, *, bq):
    nt = (((1,), (1,)), ((), ()))
    att = jax.lax.dot_general(q_ref[...], key_ref[...], nt,
                              preferred_element_type=jnp.float32) * _ISQ
    sims = jax.lax.dot_general(peq_ref[...], pekv_ref[...], nt,
                               preferred_element_type=jnp.float32)
    u = jax.lax.bitcast_convert_type(sims, jnp.int32)
    g = jnp.where(u >= 0, u, u ^ jnp.int32(0x7FFFFFFF))

    lo0 = jnp.min(g, axis=1, keepdims=True)
    hi0 = jnp.max(g, axis=1, keepdims=True) + 1
    cl0 = jnp.full((bq, 1), KNOWN, jnp.int32)

    def cond(c):
        lo, hi, cl = c
        return jnp.any((hi > lo + 1) & (cl != TOP_K))

    def body(c):
        lo, hi, cl = c
        mid = (lo & hi) + ((lo ^ hi) >> 1)
        cnt = jnp.sum(jnp.where(g >= mid, 1, 0).astype(jnp.int32),
                      axis=1, keepdims=True)
        pred = cnt >= TOP_K
        return (jnp.where(pred, mid, lo),
                jnp.where(pred, hi, mid),
                jnp.where(pred, cnt, cl))

    lo, _, _ = jax.lax.while_loop(cond, body, (lo0, hi0, cl0))
    out_ref[...] = jnp.where(g >= lo, att, 0.0)


def _attention(q, pe_q, key, pe_kv, bq):
    nq = q.shape[0]
    grid = (nq // bq,)
    import functools
    body = functools.partial(_att_body, bq=bq)
    return pl.pallas_call(
        body,
        grid=grid,
        in_specs=[
            pl.BlockSpec((bq, D), lambda i: (i, 0)),
            pl.BlockSpec((bq, D), lambda i: (i, 0)),
            pl.BlockSpec((KNOWN, D), lambda i: (0, 0)),
            pl.BlockSpec((KNOWN, D), lambda i: (0, 0)),
        ],
        out_specs=pl.BlockSpec((bq, KNOWN), lambda i: (i, 0)),
        out_shape=jax.ShapeDtypeStruct((nq, KNOWN), jnp.float32),
    )(q, pe_q, key, pe_kv)


def _metapn_body(pe_ref, cd_ref,
                 w1t, w1b, b1wt, b1b,
                 w2t, w2bm, b2wt, b2b,
                 w3pt, w3bp, b3wt, b3b, a_vec,
                 out_ref, *, bm):
    def dot(a, b):
        return jnp.dot(a, b, preferred_element_type=jnp.float32)

    pe = pe_ref[...]
    cd = cd_ref[...]
    av = a_vec[...]
    pe_b = pe.astype(jnp.bfloat16)

    g1 = dot(pe, w1t[...]) + w1b[...]
    b1r = dot(pe, b1wt[...]) + b1b[...]
    x1 = _prelu(cd[:, 0:1] * g1[:, :D] + cd[:, 1:2] * g1[:, D:] + b1r, av)

    acc = dot(pe, b2wt[...]) + b2b[...] + dot(x1, w2bm[...])
    for kc in range(8):
        g2c = dot(pe_b, w2t[:, kc * 2048:(kc + 1) * 2048])
        for j in range(16):
            k = kc * 16 + j
            acc = acc + x1[:, k:k + 1] * g2c[:, j * D:(j + 1) * D]
    x2 = _prelu(acc, av)

    g3 = dot(pe_b, w3pt[...]) + w3bp[...]
    b3r = dot(pe, b3wt[...]) + b3b[...]
    cols = [jnp.sum(x2 * g3[:, t * D:(t + 1) * D], axis=1, keepdims=True)
            for t in range(D_TREND)]
    cols.append(jnp.zeros((bm, D - D_TREND), jnp.float32))
    out_ref[...] = jnp.concatenate(cols, axis=1) + b3r


def _metapn(pe_all, cd_pad, weights, bm):
    n = pe_all.shape[0]
    import functools
    body = functools.partial(_metapn_body, bm=bm)
    wspecs = [pl.BlockSpec(w.shape, lambda i: tuple(0 for _ in w.shape))
              for w in weights]
    return pl.pallas_call(
        body,
        grid=(n // bm,),
        in_specs=[
            pl.BlockSpec((bm, D), lambda i: (i, 0)),
            pl.BlockSpec((bm, D), lambda i: (i, 0)),
        ] + wspecs,
        out_specs=pl.BlockSpec((bm, D), lambda i: (i, 0)),
        out_shape=jax.ShapeDtypeStruct((n, D), jnp.float32),
    )(pe_all, cd_pad, *weights)


def kernel(x_know, x_unknow, pe_know, pe_unknow, coods_know, coods_unknow,
           params):
    p = params
    f32 = jnp.float32
    row = lambda v: v.reshape(1, -1).astype(f32)

    a_ar = jnp.full((1, D), p['ar_a'], f32)
    prep_w = (
        p['ar_w1'].T, p['ar_w2'].T, p['ar_w3'].T,
        row(p['ar_b1']), row(p['ar_b2']), row(p['ar_b3']), a_ar,
        p['pg_w1'].T, p['pg_w12'].T, p['pg_w2'].T, p['pg_w3'].T,
        row(p['pg_b1']), row(p['pg_b12']), row(p['pg_b2']), row(p['pg_b3']),
        row(p['pg_gamma']), row(p['pg_beta']),
        p['ss_wq'].T, p['ss_wk'].T,
    )
    q_k, k_k, q_u = _prep(x_know, pe_know, x_unknow, pe_unknow, prep_w)

    cov_know = _attention(q_k, pe_know, k_k, pe_know, bq=256)
    cov_unknow = _attention(q_u, pe_unknow, k_k, pe_know, bq=256)

    w3p = p['mp_w3w'].reshape(D, D_TREND, D).transpose(1, 0, 2).reshape(
        D_TREND * D, D)
    w3bp = p['mp_w3b'].reshape(D, D_TREND).T.reshape(1, D_TREND * D)
    b3wt = jnp.zeros((D, D), f32).at[:, :D_TREND].set(p['mp_b3w'].T)
    b3b = jnp.zeros((1, D), f32).at[:, :D_TREND].set(p['mp_b3b'].reshape(1, -1))
    a_mp = jnp.full((1, D), p['mp_a'], f32)
    mp_w = (
        p['mp_w1w'].T, row(p['mp_w1b']),
        p['mp_b1w'].T, row(p['mp_b1b']),
        p['mp_w2w'].T.astype(jnp.bfloat16), p['mp_w2b'].reshape(D, D),
        p['mp_b2w'].T, row(p['mp_b2b']),
        w3p.T.astype(jnp.bfloat16), w3bp,
        b3wt, b3b, a_mp,
    )
    pe_all = jnp.concatenate([pe_know, pe_unknow], axis=0)
    cd_all = jnp.concatenate([coods_know, coods_unknow], axis=0)
    cd_pad = jnp.pad(cd_all, ((0, 0), (0, D - 2)))
    trend = _metapn(pe_all, cd_pad, mp_w, bm=256)

    return (cov_know, cov_unknow,
            trend[:KNOWN, :D_TREND], trend[KNOWN:, :D_TREND])


# speedup vs baseline: 1.4769x; 1.1707x over previous
import math

import jax
import jax.numpy as jnp
from jax.experimental import pallas as pl

D = 128
KNOWN = 2048
BATCH = 1024
D_TREND = 16
TOP_K = 64
_ISQ = 1.0 / math.sqrt(D)
_I32MIN = -2147483648
_I32MAX = 2147483647


def _prelu(x, a_vec):
    return jnp.maximum(x, 0.0) + a_vec * jnp.minimum(x, 0.0)


def _prep_body(x_k, pe_k, x_u, pe_u,
               arw1, arw2, arw3, ab1, ab2, ab3, a_ar,
               pgw1, pgw12, pgw2, pgw3, pb1, pb12, pb2, pb3, gamma, beta,
               wq, wk,
               q_k_out, k_k_out, q_u_out):
    def dot(a, b):
        return jnp.dot(a, b, preferred_element_type=jnp.float32)

    def group(x_ref, pe_ref):
        x = x_ref[...]
        pe = pe_ref[...]
        h = _prelu(dot(x, arw1[...]) + ab1[...], a_ar[...])
        h = _prelu(dot(h, arw2[...]) + ab2[...], a_ar[...])
        h = dot(h, arw3[...]) + ab3[...]
        t1 = dot(h, pgw1[...]) + pb1[...] + dot(pe, pgw12[...]) + pb12[...]
        z = (dot(t1, pgw2[...]) + pb2[...]) * jax.nn.sigmoid(
            dot(t1, pgw3[...]) + pb3[...]) + h
        m = jnp.mean(z, axis=0, keepdims=True)
        v = jnp.mean((z - m) ** 2, axis=0, keepdims=True)
        ae = gamma[...] * (z - m) / jnp.sqrt(v + 1e-5) + beta[...]
        inp = 0.5 * ae + 0.5 * pe
        return inp

    in_k = group(x_k, pe_k)
    q_k_out[...] = dot(in_k, wq[...]) + in_k
    k_k_out[...] = dot(in_k, wk[...]) + in_k
    in_u = group(x_u, pe_u)
    q_u_out[...] = dot(in_u, wq[...]) + in_u


def _prep(x_k, pe_k, x_u, pe_u, weights):
    outs = [
        jax.ShapeDtypeStruct((KNOWN, D), jnp.float32),
        jax.ShapeDtypeStruct((KNOWN, D), jnp.float32),
        jax.ShapeDtypeStruct((BATCH, D), jnp.float32),
    ]
    return pl.pallas_call(
        _prep_body,
        out_shape=outs,
    )(x_k, pe_k, x_u, pe_u, *weights)


def _att_body(q_ref, peq_ref, key_ref, pekv_ref, out_ref, *, bq):
    nt = (((1,), (1,)), ((), ()))
    att = jax.lax.dot_general(q_ref[...], key_ref[...], nt,
                              preferred_element_type=jnp.float32) * _ISQ
    sims = jax.lax.dot_general(peq_ref[...], pekv_ref[...], nt,
                               preferred_element_type=jnp.float32)
    u = jax.lax.bitcast_convert_type(sims, jnp.int32)
    g = jnp.where(u >= 0, u, u ^ jnp.int32(0x7FFFFFFF))

    def to_sortable(x):
        b = jax.lax.bitcast_convert_type(x, jnp.int32)
        return jnp.where(b >= 0, b, b ^ jnp.int32(0x7FFFFFFF))

    def count_ge(mid):
        return jnp.sum(jnp.where(g >= mid, 1, 0).astype(jnp.int32),
                       axis=1, keepdims=True)

    lo = jnp.min(g, axis=1, keepdims=True)
    hi = jnp.max(g, axis=1, keepdims=True) + 1
    cl = jnp.full((bq, 1), KNOWN, jnp.int32)

    mu = jnp.mean(sims, axis=1, keepdims=True)
    sg = jnp.sqrt(jnp.maximum(
        jnp.mean(sims * sims, axis=1, keepdims=True) - mu * mu, 0.0))
    for zval in (1.6447, 2.0807):
        piv = to_sortable(mu + zval * sg)
        piv = jnp.minimum(jnp.maximum(piv, lo + 1), hi - 1)
        valid = hi > lo + 1
        cnt = count_ge(piv)
        pred = cnt >= TOP_K
        lo = jnp.where(valid & pred, piv, lo)
        hi = jnp.where(valid & ~pred, piv, hi)
        cl = jnp.where(valid & pred, cnt, cl)

    def cond(c):
        lo, hi, cl = c
        return jnp.any((hi > lo + 1) & (cl != TOP_K))

    def body(c):
        lo, hi, cl = c
        mid = (lo & hi) + ((lo ^ hi) >> 1)
        cnt = count_ge(mid)
        pred = cnt >= TOP_K
        return (jnp.where(pred, mid, lo),
                jnp.where(pred, hi, mid),
                jnp.where(pred, cnt, cl))

    lo, _, _ = jax.lax.while_loop(cond, body, (lo, hi, cl))
    out_ref[...] = jnp.where(g >= lo, att, 0.0)


def _attention(q, pe_q, key, pe_kv, bq):
    nq = q.shape[0]
    grid = (nq // bq,)
    import functools
    body = functools.partial(_att_body, bq=bq)
    return pl.pallas_call(
        body,
        grid=grid,
        in_specs=[
            pl.BlockSpec((bq, D), lambda i: (i, 0)),
            pl.BlockSpec((bq, D), lambda i: (i, 0)),
            pl.BlockSpec((KNOWN, D), lambda i: (0, 0)),
            pl.BlockSpec((KNOWN, D), lambda i: (0, 0)),
        ],
        out_specs=pl.BlockSpec((bq, KNOWN), lambda i: (i, 0)),
        out_shape=jax.ShapeDtypeStruct((nq, KNOWN), jnp.float32),
    )(q, pe_q, key, pe_kv)


def _metapn_body(pe_ref, cd_ref,
                 w1t, w1b, b1wt, b1b,
                 w2t, w2bm, b2wt, b2b,
                 w3pt, w3bp, b3wt, b3b, a_vec,
                 out_ref, *, bm):
    def dot(a, b):
        return jnp.dot(a, b, preferred_element_type=jnp.float32)

    pe = pe_ref[...]
    cd = cd_ref[...]
    av = a_vec[...]
    pe_b = pe.astype(jnp.bfloat16)

    g1 = dot(pe, w1t[...]) + w1b[...]
    b1r = dot(pe, b1wt[...]) + b1b[...]
    x1 = _prelu(cd[:, 0:1] * g1[:, :D] + cd[:, 1:2] * g1[:, D:] + b1r, av)

    acc = dot(pe, b2wt[...]) + b2b[...] + dot(x1, w2bm[...])
    for kc in range(8):
        g2c = dot(pe_b, w2t[:, kc * 2048:(kc + 1) * 2048])
        for j in range(16):
            k = kc * 16 + j
            acc = acc + x1[:, k:k + 1] * g2c[:, j * D:(j + 1) * D]
    x2 = _prelu(acc, av)

    g3 = dot(pe_b, w3pt[...]) + w3bp[...]
    b3r = dot(pe, b3wt[...]) + b3b[...]
    cols = [jnp.sum(x2 * g3[:, t * D:(t + 1) * D], axis=1, keepdims=True)
            for t in range(D_TREND)]
    cols.append(jnp.zeros((bm, D - D_TREND), jnp.float32))
    out_ref[...] = jnp.concatenate(cols, axis=1) + b3r


def _metapn(pe_all, cd_pad, weights, bm):
    n = pe_all.shape[0]
    import functools
    body = functools.partial(_metapn_body, bm=bm)
    wspecs = [pl.BlockSpec(w.shape, lambda i: tuple(0 for _ in w.shape))
              for w in weights]
    return pl.pallas_call(
        body,
        grid=(n // bm,),
        in_specs=[
            pl.BlockSpec((bm, D), lambda i: (i, 0)),
            pl.BlockSpec((bm, D), lambda i: (i, 0)),
        ] + wspecs,
        out_specs=pl.BlockSpec((bm, D), lambda i: (i, 0)),
        out_shape=jax.ShapeDtypeStruct((n, D), jnp.float32),
    )(pe_all, cd_pad, *weights)


def kernel(x_know, x_unknow, pe_know, pe_unknow, coods_know, coods_unknow,
           params):
    p = params
    f32 = jnp.float32
    row = lambda v: v.reshape(1, -1).astype(f32)

    a_ar = jnp.full((1, D), p['ar_a'], f32)
    prep_w = (
        p['ar_w1'].T, p['ar_w2'].T, p['ar_w3'].T,
        row(p['ar_b1']), row(p['ar_b2']), row(p['ar_b3']), a_ar,
        p['pg_w1'].T, p['pg_w12'].T, p['pg_w2'].T, p['pg_w3'].T,
        row(p['pg_b1']), row(p['pg_b12']), row(p['pg_b2']), row(p['pg_b3']),
        row(p['pg_gamma']), row(p['pg_beta']),
        p['ss_wq'].T, p['ss_wk'].T,
    )
    q_k, k_k, q_u = _prep(x_know, pe_know, x_unknow, pe_unknow, prep_w)

    cov_know = _attention(q_k, pe_know, k_k, pe_know, bq=256)
    cov_unknow = _attention(q_u, pe_unknow, k_k, pe_know, bq=256)

    w3p = p['mp_w3w'].reshape(D, D_TREND, D).transpose(1, 0, 2).reshape(
        D_TREND * D, D)
    w3bp = p['mp_w3b'].reshape(D, D_TREND).T.reshape(1, D_TREND * D)
    b3wt = jnp.zeros((D, D), f32).at[:, :D_TREND].set(p['mp_b3w'].T)
    b3b = jnp.zeros((1, D), f32).at[:, :D_TREND].set(p['mp_b3b'].reshape(1, -1))
    a_mp = jnp.full((1, D), p['mp_a'], f32)
    mp_w = (
        p['mp_w1w'].T, row(p['mp_w1b']),
        p['mp_b1w'].T, row(p['mp_b1b']),
        p['mp_w2w'].T.astype(jnp.bfloat16), p['mp_w2b'].reshape(D, D),
        p['mp_b2w'].T, row(p['mp_b2b']),
        w3p.T.astype(jnp.bfloat16), w3bp,
        b3wt, b3b, a_mp,
    )
    pe_all = jnp.concatenate([pe_know, pe_unknow], axis=0)
    cd_all = jnp.concatenate([coods_know, coods_unknow], axis=0)
    cd_pad = jnp.pad(cd_all, ((0, 0), (0, D - 2)))
    trend = _metapn(pe_all, cd_pad, mp_w, bm=256)

    return (cov_know, cov_unknow,
            trend[:KNOWN, :D_TREND], trend[KNOWN:, :D_TREND])


# speedup vs baseline: 1.6662x; 1.1282x over previous
import math

import jax
import jax.numpy as jnp
from jax.experimental import pallas as pl

D = 128
KNOWN = 2048
BATCH = 1024
D_TREND = 16
TOP_K = 64
_ISQ = 1.0 / math.sqrt(D)
_I32MIN = -2147483648
_I32MAX = 2147483647


def _prelu(x, a_vec):
    return jnp.maximum(x, 0.0) + a_vec * jnp.minimum(x, 0.0)


def _dnt(a, w):
    return jax.lax.dot_general(a, w, (((1,), (1,)), ((), ())),
                               preferred_element_type=jnp.float32)


def _prep_body(x_k, pe_k, x_u, pe_u,
               arw1, arw2, arw3, ab1, ab2, ab3, a_ar,
               pgw1, pgw12, pgw2, pgw3, pb1, pb12, pb2, pb3, gamma, beta,
               wq, wk,
               q_k_out, k_k_out, q_u_out):
    def group(x_ref, pe_ref):
        x = x_ref[...]
        pe = pe_ref[...]
        h = _prelu(_dnt(x, arw1[...]) + ab1[...], a_ar[...])
        h = _prelu(_dnt(h, arw2[...]) + ab2[...], a_ar[...])
        h = _dnt(h, arw3[...]) + ab3[...]
        t1 = _dnt(h, pgw1[...]) + pb1[...] + _dnt(pe, pgw12[...]) + pb12[...]
        z = (_dnt(t1, pgw2[...]) + pb2[...]) * jax.nn.sigmoid(
            _dnt(t1, pgw3[...]) + pb3[...]) + h
        m = jnp.mean(z, axis=0, keepdims=True)
        v = jnp.mean((z - m) ** 2, axis=0, keepdims=True)
        ae = gamma[...] * (z - m) / jnp.sqrt(v + 1e-5) + beta[...]
        inp = 0.5 * ae + 0.5 * pe
        return inp

    in_k = group(x_k, pe_k)
    q_k_out[...] = _dnt(in_k, wq[...]) + in_k
    k_k_out[...] = _dnt(in_k, wk[...]) + in_k
    in_u = group(x_u, pe_u)
    q_u_out[...] = _dnt(in_u, wq[...]) + in_u


def _prep(x_k, pe_k, x_u, pe_u, weights):
    outs = [
        jax.ShapeDtypeStruct((KNOWN, D), jnp.float32),
        jax.ShapeDtypeStruct((KNOWN, D), jnp.float32),
        jax.ShapeDtypeStruct((BATCH, D), jnp.float32),
    ]
    return pl.pallas_call(
        _prep_body,
        out_shape=outs,
    )(x_k, pe_k, x_u, pe_u, *weights)


def _att_body(q_ref, peq_ref, key_ref, pekv_ref, out_ref, *, bq):
    nt = (((1,), (1,)), ((), ()))
    att = jax.lax.dot_general(q_ref[...], key_ref[...], nt,
                              preferred_element_type=jnp.float32) * _ISQ
    sims = jax.lax.dot_general(peq_ref[...], pekv_ref[...], nt,
                               preferred_element_type=jnp.float32)
    u = jax.lax.bitcast_convert_type(sims, jnp.int32)
    g = jnp.where(u >= 0, u, u ^ jnp.int32(0x7FFFFFFF))

    def to_sortable(x):
        b = jax.lax.bitcast_convert_type(x, jnp.int32)
        return jnp.where(b >= 0, b, b ^ jnp.int32(0x7FFFFFFF))

    def count_ge(mid):
        return jnp.sum(jnp.where(g >= mid, 1, 0).astype(jnp.int32),
                       axis=1, keepdims=True)

    lo = jnp.min(g, axis=1, keepdims=True)
    hi = jnp.max(g, axis=1, keepdims=True) + 1
    cl = jnp.full((bq, 1), KNOWN, jnp.int32)

    mu = jnp.mean(sims, axis=1, keepdims=True)
    sg = jnp.sqrt(jnp.maximum(
        jnp.mean(sims * sims, axis=1, keepdims=True) - mu * mu, 0.0))

    def unsort(v):
        return jax.lax.bitcast_convert_type(
            jnp.where(v >= 0, v, v ^ jnp.int32(0x7FFFFFFF)), jnp.float32)

    def step(lo, hi, cl, piv):
        piv = jnp.minimum(jnp.maximum(piv, lo + 1), hi - 1)
        valid = hi > lo + 1
        cnt = count_ge(piv)
        pred = cnt >= TOP_K
        return (jnp.where(valid & pred, piv, lo),
                jnp.where(valid & ~pred, piv, hi),
                jnp.where(valid & pred, cnt, cl))

    for zval in (1.6447, 2.0807):
        lo, hi, cl = step(lo, hi, cl, to_sortable(mu + zval * sg))
    for _ in range(5):
        lo, hi, cl = step(lo, hi, cl,
                          to_sortable(0.5 * (unsort(lo) + unsort(hi))))

    def cond(c):
        lo, hi, cl = c
        return jnp.any((hi > lo + 1) & (cl != TOP_K))

    def body(c):
        lo, hi, cl = c
        mid = (lo & hi) + ((lo ^ hi) >> 1)
        cnt = count_ge(mid)
        pred = cnt >= TOP_K
        return (jnp.where(pred, mid, lo),
                jnp.where(pred, hi, mid),
                jnp.where(pred, cnt, cl))

    lo, _, _ = jax.lax.while_loop(cond, body, (lo, hi, cl))
    out_ref[...] = jnp.where(g >= lo, att, 0.0)


def _attention(q, pe_q, key, pe_kv, bq):
    nq = q.shape[0]
    grid = (nq // bq,)
    import functools
    body = functools.partial(_att_body, bq=bq)
    return pl.pallas_call(
        body,
        grid=grid,
        in_specs=[
            pl.BlockSpec((bq, D), lambda i: (i, 0)),
            pl.BlockSpec((bq, D), lambda i: (i, 0)),
            pl.BlockSpec((KNOWN, D), lambda i: (0, 0)),
            pl.BlockSpec((KNOWN, D), lambda i: (0, 0)),
        ],
        out_specs=pl.BlockSpec((bq, KNOWN), lambda i: (i, 0)),
        out_shape=jax.ShapeDtypeStruct((nq, KNOWN), jnp.float32),
    )(q, pe_q, key, pe_kv)


def _metapn_body(pe_ref, cd_ref,
                 w1t, w1b, b1wt, b1b,
                 w2t, w2bm, b2wt, b2b,
                 w3pt, w3bp, b3wt, b3b, a_vec,
                 out_ref, *, bm):
    pe = pe_ref[...]
    cd = cd_ref[...]
    av = a_vec[...]
    pe_b = pe.astype(jnp.bfloat16)

    g1 = _dnt(pe, w1t[...]) + w1b[...]
    b1r = _dnt(pe, b1wt[...]) + b1b[...]
    x1 = _prelu(cd[:, 0:1] * g1[:, :D] + cd[:, 1:2] * g1[:, D:] + b1r, av)

    acc = _dnt(pe, b2wt[...]) + b2b[...] + jnp.dot(
        x1, w2bm[...], preferred_element_type=jnp.float32)
    for kc in range(8):
        g2c = _dnt(pe_b, w2t[kc * 2048:(kc + 1) * 2048, :])
        for j in range(16):
            k = kc * 16 + j
            acc = acc + x1[:, k:k + 1] * g2c[:, j * D:(j + 1) * D]
    x2 = _prelu(acc, av)

    g3 = _dnt(pe_b, w3pt[...]) + w3bp[...]
    b3r = _dnt(pe, b3wt[...]) + b3b[...]
    cols = [jnp.sum(x2 * g3[:, t * D:(t + 1) * D], axis=1, keepdims=True)
            for t in range(D_TREND)]
    cols.append(jnp.zeros((bm, D - D_TREND), jnp.float32))
    out_ref[...] = jnp.concatenate(cols, axis=1) + b3r


def _metapn(pe_all, cd_pad, weights, bm):
    n = pe_all.shape[0]
    import functools
    body = functools.partial(_metapn_body, bm=bm)
    wspecs = [pl.BlockSpec(w.shape, lambda i: tuple(0 for _ in w.shape))
              for w in weights]
    return pl.pallas_call(
        body,
        grid=(n // bm,),
        in_specs=[
            pl.BlockSpec((bm, D), lambda i: (i, 0)),
            pl.BlockSpec((bm, D), lambda i: (i, 0)),
        ] + wspecs,
        out_specs=pl.BlockSpec((bm, D), lambda i: (i, 0)),
        out_shape=jax.ShapeDtypeStruct((n, D), jnp.float32),
    )(pe_all, cd_pad, *weights)


def kernel(x_know, x_unknow, pe_know, pe_unknow, coods_know, coods_unknow,
           params):
    p = params
    f32 = jnp.float32
    row = lambda v: v.reshape(1, -1).astype(f32)

    a_ar = jnp.full((1, D), p['ar_a'], f32)
    prep_w = (
        p['ar_w1'], p['ar_w2'], p['ar_w3'],
        row(p['ar_b1']), row(p['ar_b2']), row(p['ar_b3']), a_ar,
        p['pg_w1'], p['pg_w12'], p['pg_w2'], p['pg_w3'],
        row(p['pg_b1']), row(p['pg_b12']), row(p['pg_b2']), row(p['pg_b3']),
        row(p['pg_gamma']), row(p['pg_beta']),
        p['ss_wq'], p['ss_wk'],
    )
    q_k, k_k, q_u = _prep(x_know, pe_know, x_unknow, pe_unknow, prep_w)

    cov_know = _attention(q_k, pe_know, k_k, pe_know, bq=256)
    cov_unknow = _attention(q_u, pe_unknow, k_k, pe_know, bq=256)

    w3p = p['mp_w3w'].reshape(D, D_TREND, D).transpose(1, 0, 2).reshape(
        D_TREND * D, D)
    w3bp = p['mp_w3b'].reshape(D, D_TREND).T.reshape(1, D_TREND * D)
    b3wp = jnp.zeros((D, D), f32).at[:D_TREND, :].set(p['mp_b3w'])
    b3b = jnp.zeros((1, D), f32).at[:, :D_TREND].set(p['mp_b3b'].reshape(1, -1))
    a_mp = jnp.full((1, D), p['mp_a'], f32)
    mp_w = (
        p['mp_w1w'], row(p['mp_w1b']),
        p['mp_b1w'], row(p['mp_b1b']),
        p['mp_w2w'].astype(jnp.bfloat16), p['mp_w2b'].reshape(D, D),
        p['mp_b2w'], row(p['mp_b2b']),
        w3p.astype(jnp.bfloat16), w3bp,
        b3wp, b3b, a_mp,
    )
    pe_all = jnp.concatenate([pe_know, pe_unknow], axis=0)
    cd_all = jnp.concatenate([coods_know, coods_unknow], axis=0)
    cd_pad = jnp.pad(cd_all, ((0, 0), (0, D - 2)))
    trend = _metapn(pe_all, cd_pad, mp_w, bm=256)

    return (cov_know, cov_unknow,
            trend[:KNOWN, :D_TREND], trend[KNOWN:, :D_TREND])


# speedup vs baseline: 1.6927x; 1.0159x over previous
import math

import jax
import jax.numpy as jnp
from jax.experimental import pallas as pl
from jax.experimental.pallas import tpu as pltpu

D = 128
KNOWN = 2048
BATCH = 1024
D_TREND = 16
TOP_K = 64
_ISQ = 1.0 / math.sqrt(D)
_I32MIN = -2147483648
_I32MAX = 2147483647


def _prelu(x, a_vec):
    return jnp.maximum(x, 0.0) + a_vec * jnp.minimum(x, 0.0)


def _dnt(a, w):
    return jax.lax.dot_general(a, w, (((1,), (1,)), ((), ())),
                               preferred_element_type=jnp.float32)


def _prep_body(x_k, pe_k, x_u, pe_u,
               arw1, arw2, arw3, ab1, ab2, ab3, a_ar,
               pgw1, pgw12, pgw2, pgw3, pb1, pb12, pb2, pb3, gamma, beta,
               wq, wk,
               q_k_out, k_k_out, q_u_out):
    def group(x_ref, pe_ref):
        x = x_ref[...]
        pe = pe_ref[...]
        h = _prelu(_dnt(x, arw1[...]) + ab1[...], a_ar[...])
        h = _prelu(_dnt(h, arw2[...]) + ab2[...], a_ar[...])
        h = _dnt(h, arw3[...]) + ab3[...]
        t1 = _dnt(h, pgw1[...]) + pb1[...] + _dnt(pe, pgw12[...]) + pb12[...]
        z = (_dnt(t1, pgw2[...]) + pb2[...]) * jax.nn.sigmoid(
            _dnt(t1, pgw3[...]) + pb3[...]) + h
        m = jnp.mean(z, axis=0, keepdims=True)
        v = jnp.mean((z - m) ** 2, axis=0, keepdims=True)
        ae = gamma[...] * (z - m) / jnp.sqrt(v + 1e-5) + beta[...]
        inp = 0.5 * ae + 0.5 * pe
        return inp

    in_k = group(x_k, pe_k)
    q_k_out[...] = _dnt(in_k, wq[...]) + in_k
    k_k_out[...] = _dnt(in_k, wk[...]) + in_k
    in_u = group(x_u, pe_u)
    q_u_out[...] = _dnt(in_u, wq[...]) + in_u


def _prep(x_k, pe_k, x_u, pe_u, weights):
    outs = [
        jax.ShapeDtypeStruct((KNOWN, D), jnp.float32),
        jax.ShapeDtypeStruct((KNOWN, D), jnp.float32),
        jax.ShapeDtypeStruct((BATCH, D), jnp.float32),
    ]
    return pl.pallas_call(
        _prep_body,
        out_shape=outs,
    )(x_k, pe_k, x_u, pe_u, *weights)


def _att_body(q_ref, peq_ref, key_ref, pekv_ref, out_ref, *, bq):
    nt = (((1,), (1,)), ((), ()))
    att = jax.lax.dot_general(q_ref[...], key_ref[...], nt,
                              preferred_element_type=jnp.float32) * _ISQ
    sims = jax.lax.dot_general(peq_ref[...], pekv_ref[...], nt,
                               preferred_element_type=jnp.float32)
    u = jax.lax.bitcast_convert_type(sims, jnp.int32)
    g = jnp.where(u >= 0, u, u ^ jnp.int32(0x7FFFFFFF))

    def to_sortable(x):
        b = jax.lax.bitcast_convert_type(x, jnp.int32)
        return jnp.where(b >= 0, b, b ^ jnp.int32(0x7FFFFFFF))

    def count_ge(mid):
        return jnp.sum(jnp.where(g >= mid, 1, 0).astype(jnp.int32),
                       axis=1, keepdims=True)

    lo = jnp.min(g, axis=1, keepdims=True)
    hi = jnp.max(g, axis=1, keepdims=True) + 1
    cl = jnp.full((bq, 1), KNOWN, jnp.int32)

    mu = jnp.mean(sims, axis=1, keepdims=True)
    sg = jnp.sqrt(jnp.maximum(
        jnp.mean(sims * sims, axis=1, keepdims=True) - mu * mu, 0.0))

    def unsort(v):
        return jax.lax.bitcast_convert_type(
            jnp.where(v >= 0, v, v ^ jnp.int32(0x7FFFFFFF)), jnp.float32)

    def step(lo, hi, cl, piv):
        piv = jnp.minimum(jnp.maximum(piv, lo + 1), hi - 1)
        valid = hi > lo + 1
        cnt = count_ge(piv)
        pred = cnt >= TOP_K
        return (jnp.where(valid & pred, piv, lo),
                jnp.where(valid & ~pred, piv, hi),
                jnp.where(valid & pred, cnt, cl))

    for zval in (1.6447, 2.0807):
        lo, hi, cl = step(lo, hi, cl, to_sortable(mu + zval * sg))
    for _ in range(5):
        lo, hi, cl = step(lo, hi, cl,
                          to_sortable(0.5 * (unsort(lo) + unsort(hi))))

    def cond(c):
        lo, hi, cl = c
        return jnp.any((hi > lo + 1) & (cl != TOP_K))

    def body(c):
        lo, hi, cl = c
        mid = (lo & hi) + ((lo ^ hi) >> 1)
        cnt = count_ge(mid)
        pred = cnt >= TOP_K
        return (jnp.where(pred, mid, lo),
                jnp.where(pred, hi, mid),
                jnp.where(pred, cnt, cl))

    lo, _, _ = jax.lax.while_loop(cond, body, (lo, hi, cl))
    out_ref[...] = jnp.where(g >= lo, att, 0.0)


def _attention(q, pe_q, key, pe_kv, bq):
    nq = q.shape[0]
    grid = (nq // bq,)
    import functools
    body = functools.partial(_att_body, bq=bq)
    return pl.pallas_call(
        body,
        grid=grid,
        in_specs=[
            pl.BlockSpec((bq, D), lambda i: (i, 0)),
            pl.BlockSpec((bq, D), lambda i: (i, 0)),
            pl.BlockSpec((KNOWN, D), lambda i: (0, 0)),
            pl.BlockSpec((KNOWN, D), lambda i: (0, 0)),
        ],
        out_specs=pl.BlockSpec((bq, KNOWN), lambda i: (i, 0)),
        out_shape=jax.ShapeDtypeStruct((nq, KNOWN), jnp.float32),
    )(q, pe_q, key, pe_kv)


def _metapn_body(pe_ref, cd_ref,
                 w1t, w1b, b1wt, b1b,
                 w2t, w2bm, b2wt, b2b,
                 w3pt, w3bp, b3wt, b3b, a_vec,
                 out_ref, *, bm):
    pe = pe_ref[...]
    cd = cd_ref[...]
    av = a_vec[...]
    pe_b = pe.astype(jnp.bfloat16)

    g1 = _dnt(pe, w1t[...]) + w1b[...]
    b1r = _dnt(pe, b1wt[...]) + b1b[...]
    x1 = _prelu(cd[:, 0:1] * g1[:, :D] + cd[:, 1:2] * g1[:, D:] + b1r, av)

    acc = _dnt(pe, b2wt[...]) + b2b[...] + jnp.dot(
        x1, w2bm[...], preferred_element_type=jnp.float32)
    for kc in range(8):
        g2c = _dnt(pe_b, w2t[kc * 2048:(kc + 1) * 2048, :])
        for j in range(16):
            k = kc * 16 + j
            acc = acc + x1[:, k:k + 1] * g2c[:, j * D:(j + 1) * D]
    x2 = _prelu(acc, av)

    g3 = _dnt(pe_b, w3pt[...]) + w3bp[...]
    b3r = _dnt(pe, b3wt[...]) + b3b[...]
    cols = [jnp.sum(x2 * g3[:, t * D:(t + 1) * D], axis=1, keepdims=True)
            for t in range(D_TREND)]
    cols.append(jnp.zeros((bm, D - D_TREND), jnp.float32))
    out_ref[...] = jnp.concatenate(cols, axis=1) + b3r


def _metapn(pe_all, cd_pad, weights, bm):
    n = pe_all.shape[0]
    import functools
    body = functools.partial(_metapn_body, bm=bm)
    wspecs = [pl.BlockSpec(w.shape, lambda i: tuple(0 for _ in w.shape))
              for w in weights]
    return pl.pallas_call(
        body,
        grid=(n // bm,),
        in_specs=[
            pl.BlockSpec((bm, D), lambda i: (i, 0)),
            pl.BlockSpec((bm, D), lambda i: (i, 0)),
        ] + wspecs,
        out_specs=pl.BlockSpec((bm, D), lambda i: (i, 0)),
        out_shape=jax.ShapeDtypeStruct((n, D), jnp.float32),
    )(pe_all, cd_pad, *weights)


def kernel(x_know, x_unknow, pe_know, pe_unknow, coods_know, coods_unknow,
           params):
    p = params
    f32 = jnp.float32
    row = lambda v: v.reshape(1, -1).astype(f32)

    a_ar = jnp.full((1, D), p['ar_a'], f32)
    prep_w = (
        p['ar_w1'], p['ar_w2'], p['ar_w3'],
        row(p['ar_b1']), row(p['ar_b2']), row(p['ar_b3']), a_ar,
        p['pg_w1'], p['pg_w12'], p['pg_w2'], p['pg_w3'],
        row(p['pg_b1']), row(p['pg_b12']), row(p['pg_b2']), row(p['pg_b3']),
        row(p['pg_gamma']), row(p['pg_beta']),
        p['ss_wq'], p['ss_wk'],
    )
    q_k, k_k, q_u = _prep(x_know, pe_know, x_unknow, pe_unknow, prep_w)

    cov_know = _attention(q_k, pe_know, k_k, pe_know, bq=512)
    cov_unknow = _attention(q_u, pe_unknow, k_k, pe_know, bq=512)

    w3p = p['mp_w3w'].reshape(D, D_TREND, D).transpose(1, 0, 2).reshape(
        D_TREND * D, D)
    w3bp = p['mp_w3b'].reshape(D, D_TREND).T.reshape(1, D_TREND * D)
    b3wp = jnp.zeros((D, D), f32).at[:D_TREND, :].set(p['mp_b3w'])
    b3b = jnp.zeros((1, D), f32).at[:, :D_TREND].set(p['mp_b3b'].reshape(1, -1))
    a_mp = jnp.full((1, D), p['mp_a'], f32)
    mp_w = (
        p['mp_w1w'], row(p['mp_w1b']),
        p['mp_b1w'], row(p['mp_b1b']),
        p['mp_w2w'].astype(jnp.bfloat16), p['mp_w2b'].reshape(D, D),
        p['mp_b2w'], row(p['mp_b2b']),
        w3p.astype(jnp.bfloat16), w3bp,
        b3wp, b3b, a_mp,
    )
    pe_all = jnp.concatenate([pe_know, pe_unknow], axis=0)
    cd_all = jnp.concatenate([coods_know, coods_unknow], axis=0)
    cd_pad = jnp.pad(cd_all, ((0, 0), (0, D - 2)))
    trend = _metapn(pe_all, cd_pad, mp_w, bm=256)

    return (cov_know, cov_unknow,
            trend[:KNOWN, :D_TREND], trend[KNOWN:, :D_TREND])


# speedup vs baseline: 1.6954x; 1.0016x over previous
import math

import functools

import jax
import jax.numpy as jnp
from jax.experimental import pallas as pl

D = 128
KNOWN = 2048
BATCH = 1024
D_TREND = 16
TOP_K = 64
_ISQ = 1.0 / math.sqrt(D)


def _prelu(x, a_vec):
    return jnp.maximum(x, 0.0) + a_vec * jnp.minimum(x, 0.0)


def _dnt(a, w):
    return jax.lax.dot_general(a, w, (((1,), (1,)), ((), ())),
                               preferred_element_type=jnp.float32)


def _prep_body(x_k, pe_k, x_u, pe_u,
               arw1, arw2, arw3, ab1, ab2, ab3, a_ar,
               pgw1, pgw12, pgw2, pgw3, pb1, pb12, pb2, pb3, gamma, beta,
               wq, wk,
               q_k_out, k_k_out, q_u_out):
    def group(x_ref, pe_ref):
        x = x_ref[...]
        pe = pe_ref[...]
        h = _prelu(_dnt(x, arw1[...]) + ab1[...], a_ar[...])
        h = _prelu(_dnt(h, arw2[...]) + ab2[...], a_ar[...])
        h = _dnt(h, arw3[...]) + ab3[...]
        t1 = _dnt(h, pgw1[...]) + pb1[...] + _dnt(pe, pgw12[...]) + pb12[...]
        z = (_dnt(t1, pgw2[...]) + pb2[...]) * jax.nn.sigmoid(
            _dnt(t1, pgw3[...]) + pb3[...]) + h
        m = jnp.mean(z, axis=0, keepdims=True)
        v = jnp.mean((z - m) ** 2, axis=0, keepdims=True)
        ae = gamma[...] * (z - m) / jnp.sqrt(v + 1e-5) + beta[...]
        inp = 0.5 * ae + 0.5 * pe
        return inp

    in_k = group(x_k, pe_k)
    q_k_out[...] = _dnt(in_k, wq[...]) + in_k
    k_k_out[...] = _dnt(in_k, wk[...]) + in_k
    in_u = group(x_u, pe_u)
    q_u_out[...] = _dnt(in_u, wq[...]) + in_u


def _prep(x_k, pe_k, x_u, pe_u, weights):
    outs = [
        jax.ShapeDtypeStruct((KNOWN, D), jnp.float32),
        jax.ShapeDtypeStruct((KNOWN, D), jnp.float32),
        jax.ShapeDtypeStruct((BATCH, D), jnp.float32),
    ]
    return pl.pallas_call(
        _prep_body,
        out_shape=outs,
    )(x_k, pe_k, x_u, pe_u, *weights)


def _att_body(q_ref, peq_ref, key_ref, pekv_ref, out_ref, *, bq):
    nt = (((1,), (1,)), ((), ()))
    att = jax.lax.dot_general(q_ref[...], key_ref[...], nt,
                              preferred_element_type=jnp.float32) * _ISQ
    sims = jax.lax.dot_general(peq_ref[...], pekv_ref[...], nt,
                               preferred_element_type=jnp.float32)
    u = jax.lax.bitcast_convert_type(sims, jnp.int32)
    g = jnp.where(u >= 0, u, u ^ jnp.int32(0x7FFFFFFF))

    def to_sortable(x):
        b = jax.lax.bitcast_convert_type(x, jnp.int32)
        return jnp.where(b >= 0, b, b ^ jnp.int32(0x7FFFFFFF))

    def count_ge(mid):
        return jnp.sum(jnp.where(g >= mid, 1, 0).astype(jnp.int32),
                       axis=1, keepdims=True)

    lo = jnp.min(g, axis=1, keepdims=True)
    hi = jnp.max(g, axis=1, keepdims=True) + 1
    cl = jnp.full((bq, 1), KNOWN, jnp.int32)

    mu = jnp.mean(sims, axis=1, keepdims=True)
    sg = jnp.sqrt(jnp.maximum(
        jnp.mean(sims * sims, axis=1, keepdims=True) - mu * mu, 0.0))

    def unsort(v):
        return jax.lax.bitcast_convert_type(
            jnp.where(v >= 0, v, v ^ jnp.int32(0x7FFFFFFF)), jnp.float32)

    def step(lo, hi, cl, piv):
        piv = jnp.minimum(jnp.maximum(piv, lo + 1), hi - 1)
        valid = hi > lo + 1
        cnt = count_ge(piv)
        pred = cnt >= TOP_K
        return (jnp.where(valid & pred, piv, lo),
                jnp.where(valid & ~pred, piv, hi),
                jnp.where(valid & pred, cnt, cl))

    for zval in (1.6447, 2.0807):
        lo, hi, cl = step(lo, hi, cl, to_sortable(mu + zval * sg))
    for _ in range(5):
        lo, hi, cl = step(lo, hi, cl,
                          to_sortable(0.5 * (unsort(lo) + unsort(hi))))

    def cond(c):
        lo, hi, cl = c
        return jnp.any((hi > lo + 1) & (cl != TOP_K))

    def body(c):
        lo, hi, cl = c
        mid = (lo & hi) + ((lo ^ hi) >> 1)
        cnt = count_ge(mid)
        pred = cnt >= TOP_K
        return (jnp.where(pred, mid, lo),
                jnp.where(pred, hi, mid),
                jnp.where(pred, cnt, cl))

    lo, _, _ = jax.lax.while_loop(cond, body, (lo, hi, cl))
    out_ref[...] = jnp.where(g >= lo, att, 0.0)


def _attention(q, pe_q, key, pe_kv, bq):
    nq = q.shape[0]
    grid = (nq // bq,)
    body = functools.partial(_att_body, bq=bq)
    return pl.pallas_call(
        body,
        grid=grid,
        in_specs=[
            pl.BlockSpec((bq, D), lambda i: (i, 0)),
            pl.BlockSpec((bq, D), lambda i: (i, 0)),
            pl.BlockSpec((KNOWN, D), lambda i: (0, 0)),
            pl.BlockSpec((KNOWN, D), lambda i: (0, 0)),
        ],
        out_specs=pl.BlockSpec((bq, KNOWN), lambda i: (i, 0)),
        out_shape=jax.ShapeDtypeStruct((nq, KNOWN), jnp.float32),
    )(q, pe_q, key, pe_kv)


def _metapn_body(pe_ref, cd_ref,
                 w1t, w1b, b1wt, b1b,
                 w2t, w2bm, b2wt, b2b,
                 w3pt, w3bp, b3wt, b3b, a_vec,
                 out_ref, *, bm):
    pe = pe_ref[...]
    cd = cd_ref[...]
    av = a_vec[...]
    pe_b = pe.astype(jnp.bfloat16)

    g1 = _dnt(pe, w1t[...]) + w1b[...]
    b1r = _dnt(pe, b1wt[...]) + b1b[...]
    x1 = _prelu(cd[:, 0:1] * g1[:, :D] + cd[:, 1:2] * g1[:, D:] + b1r, av)

    acc = _dnt(pe, b2wt[...]) + b2b[...] + jnp.dot(
        x1, w2bm[...], preferred_element_type=jnp.float32)
    for kc in range(8):
        g2c = _dnt(pe_b, w2t[kc * 2048:(kc + 1) * 2048, :])
        for j in range(16):
            k = kc * 16 + j
            acc = acc + x1[:, k:k + 1] * g2c[:, j * D:(j + 1) * D]
    x2 = _prelu(acc, av)

    g3 = _dnt(pe_b, w3pt[...]) + w3bp[...]
    b3r = _dnt(pe, b3wt[...]) + b3b[...]
    cols = [jnp.sum(x2 * g3[:, t * D:(t + 1) * D], axis=1, keepdims=True)
            for t in range(D_TREND)]
    cols.append(jnp.zeros((bm, D - D_TREND), jnp.float32))
    out_ref[...] = jnp.concatenate(cols, axis=1) + b3r


def _metapn(pe_all, cd_pad, weights, bm):
    n = pe_all.shape[0]
    body = functools.partial(_metapn_body, bm=bm)
    wspecs = [pl.BlockSpec(w.shape, lambda i: tuple(0 for _ in w.shape))
              for w in weights]
    return pl.pallas_call(
        body,
        grid=(n // bm,),
        in_specs=[
            pl.BlockSpec((bm, D), lambda i: (i, 0)),
            pl.BlockSpec((bm, D), lambda i: (i, 0)),
        ] + wspecs,
        out_specs=pl.BlockSpec((bm, D), lambda i: (i, 0)),
        out_shape=jax.ShapeDtypeStruct((n, D), jnp.float32),
    )(pe_all, cd_pad, *weights)


def kernel(x_know, x_unknow, pe_know, pe_unknow, coods_know, coods_unknow,
           params):
    p = params
    f32 = jnp.float32
    row = lambda v: v.reshape(1, -1).astype(f32)

    a_ar = jnp.full((1, D), p['ar_a'], f32)
    prep_w = (
        p['ar_w1'], p['ar_w2'], p['ar_w3'],
        row(p['ar_b1']), row(p['ar_b2']), row(p['ar_b3']), a_ar,
        p['pg_w1'], p['pg_w12'], p['pg_w2'], p['pg_w3'],
        row(p['pg_b1']), row(p['pg_b12']), row(p['pg_b2']), row(p['pg_b3']),
        row(p['pg_gamma']), row(p['pg_beta']),
        p['ss_wq'], p['ss_wk'],
    )
    q_k, k_k, q_u = _prep(x_know, pe_know, x_unknow, pe_unknow, prep_w)

    cov_know = _attention(q_k, pe_know, k_k, pe_know, bq=512)
    cov_unknow = _attention(q_u, pe_unknow, k_k, pe_know, bq=512)

    w3p = p['mp_w3w'].reshape(D, D_TREND, D).transpose(1, 0, 2).reshape(
        D_TREND * D, D)
    w3bp = p['mp_w3b'].reshape(D, D_TREND).T.reshape(1, D_TREND * D)
    b3wp = jnp.zeros((D, D), f32).at[:D_TREND, :].set(p['mp_b3w'])
    b3b = jnp.zeros((1, D), f32).at[:, :D_TREND].set(p['mp_b3b'].reshape(1, -1))
    a_mp = jnp.full((1, D), p['mp_a'], f32)
    mp_w = (
        p['mp_w1w'], row(p['mp_w1b']),
        p['mp_b1w'], row(p['mp_b1b']),
        p['mp_w2w'].astype(jnp.bfloat16), p['mp_w2b'].reshape(D, D),
        p['mp_b2w'], row(p['mp_b2b']),
        w3p.astype(jnp.bfloat16), w3bp,
        b3wp, b3b, a_mp,
    )
    pe_all = jnp.concatenate([pe_know, pe_unknow], axis=0)
    cd_all = jnp.concatenate([coods_know, coods_unknow], axis=0)
    cd_pad = jnp.pad(cd_all, ((0, 0), (0, D - 2)))
    trend = _metapn(pe_all, cd_pad, mp_w, bm=256)

    return (cov_know, cov_unknow,
            trend[:KNOWN, :D_TREND], trend[KNOWN:, :D_TREND])
